# Initial kernel scaffold; baseline (speedup 1.0000x reference)
import jax
import jax.numpy as jnp
from jax.experimental import pallas as pl


def kernel(x, edge_index, train_mask, W, b):
    raise NotImplementedError("write your pallas kernel here")


# speedup vs baseline: 13.0527x; 13.0527x over previous
import functools

import jax
import jax.numpy as jnp
from jax import lax
from jax.experimental import pallas as pl
from jax.experimental.pallas import tpu as pltpu
from jax.experimental.pallas import tpu_sc as plsc

NC = 2
NS = 16
NW = NC * NS
LANES = 16
K = 128
N_PAD = 10112
IDXBITS = 14
BLK = 128
DT = 136


def _mesh():
    return plsc.VectorSubcoreMesh(
        core_axis_name="c", subcore_axis_name="s",
        num_cores=NC, num_subcores=NS)


def _sc_params():
    return pltpu.CompilerParams(use_tc_tiling_on_sc=False)


def _unpack_loop(rc_v, row_v, col_v, nchunk):
    def body(j, carry):
        for q in range(K // LANES):
            sl = pl.ds(q * LANES, LANES)
            v = rc_v[j, sl]
            row_v[j, sl] = lax.shift_right_logical(v, IDXBITS)
            col_v[j, sl] = lax.bitwise_and(v, (1 << IDXBITS) - 1)
        return carry
    lax.fori_loop(0, nchunk, body, 0)


def _sc_degree(rc3, n_pad):
    nchunk = rc3.shape[1]
    rps = n_pad // NS

    @functools.partial(
        pl.kernel,
        out_type=jax.ShapeDtypeStruct((NC, n_pad), jnp.float32),
        mesh=_mesh(),
        compiler_params=_sc_params(),
        scratch_types=[
            pltpu.VMEM((nchunk, K), jnp.int32),
            pltpu.VMEM((nchunk, K), jnp.int32),
            pltpu.VMEM((K,), jnp.float32),
            pltpu.VMEM_SHARED((n_pad,), jnp.float32),
        ],
    )
    def deg_kernel(rc_hbm, ones_hbm, zeros_hbm, deg_hbm,
                   rc_v, row_v, ones_v, deg_sp):
        c = lax.axis_index("c")
        s = lax.axis_index("s")
        t = c * NS + s

        pltpu.sync_copy(ones_hbm, ones_v)
        pltpu.sync_copy(zeros_hbm, deg_sp.at[pl.ds(s * rps, rps)])
        pltpu.sync_copy(rc_hbm.at[t], rc_v)
        _unpack_loop(rc_v, row_v, rc_v, nchunk)
        plsc.subcore_barrier()

        def body(j, carry):
            pltpu.sync_copy(ones_v, deg_sp.at[row_v.at[j]], add=True)
            return carry
        lax.fori_loop(0, nchunk, body, 0)

        plsc.subcore_barrier()
        sl = pl.ds(s * rps, rps)
        pltpu.sync_copy(deg_sp.at[sl], deg_hbm.at[c].at[sl])

    return deg_kernel(rc3, jnp.ones((K,), jnp.float32),
                      jnp.zeros((rps,), jnp.float32))


def _sc_spmm(z, dis1, dm1, rc3, n_pad, d):
    nchunk = rc3.shape[1]
    rps = n_pad // NS

    @functools.partial(
        pl.kernel,
        out_type=[
            jax.ShapeDtypeStruct((NC, n_pad, d), jnp.float32),
            jax.ShapeDtypeStruct((NC, n_pad), jnp.float32),
            jax.ShapeDtypeStruct((NC, n_pad), jnp.float32),
        ],
        mesh=_mesh(),
        compiler_params=_sc_params(),
        scratch_types=[
            pltpu.VMEM((nchunk, K), jnp.int32),
            pltpu.VMEM((nchunk, K), jnp.int32),
            pltpu.VMEM((nchunk, K), jnp.int32),
            pltpu.VMEM((K, d), jnp.float32),
            pltpu.VMEM((K,), jnp.float32),
            pltpu.VMEM((K,), jnp.float32),
            pltpu.VMEM_SHARED((n_pad, d), jnp.float32),
            pltpu.VMEM_SHARED((n_pad,), jnp.float32),
            pltpu.VMEM_SHARED((n_pad,), jnp.float32),
        ],
    )
    def spmm_kernel(z_hbm, dis_hbm, dm_hbm, rc_hbm, zeros_hbm, zeros1_hbm,
                    acc_hbm, ax_hbm, cx_hbm,
                    rc_v, row_v, col_v, gbuf, abuf, cbuf,
                    acc_sp, a_sp, c_sp):
        c = lax.axis_index("c")
        s = lax.axis_index("s")
        t = c * NS + s

        pltpu.sync_copy(zeros_hbm, acc_sp.at[pl.ds(s * rps, rps)])
        pltpu.sync_copy(zeros1_hbm, a_sp.at[pl.ds(s * rps, rps)])
        pltpu.sync_copy(zeros1_hbm, c_sp.at[pl.ds(s * rps, rps)])
        pltpu.sync_copy(rc_hbm.at[t], rc_v)
        _unpack_loop(rc_v, row_v, col_v, nchunk)
        plsc.subcore_barrier()

        def body(j, carry):
            pltpu.sync_copy(z_hbm.at[col_v.at[j]], gbuf)
            pltpu.sync_copy(dis_hbm.at[col_v.at[j]], abuf)
            pltpu.sync_copy(dm_hbm.at[col_v.at[j]], cbuf)
            pltpu.sync_copy(gbuf, acc_sp.at[row_v.at[j]], add=True)
            pltpu.sync_copy(abuf, a_sp.at[row_v.at[j]], add=True)
            pltpu.sync_copy(cbuf, c_sp.at[row_v.at[j]], add=True)
            return carry
        lax.fori_loop(0, nchunk, body, 0)

        plsc.subcore_barrier()
        sl = pl.ds(s * rps, rps)
        pltpu.sync_copy(acc_sp.at[sl], acc_hbm.at[c].at[sl])
        pltpu.sync_copy(a_sp.at[sl], ax_hbm.at[c].at[sl])
        pltpu.sync_copy(c_sp.at[sl], cx_hbm.at[c].at[sl])

    return spmm_kernel(z, dis1, dm1, rc3,
                       jnp.zeros((rps, d), jnp.float32),
                       jnp.zeros((rps,), jnp.float32))


def _tc_build_z(x_pad, m_pad, deg0, deg1, n_pad, d):
    def zk(x_ref, m_ref, d0_ref, d1_ref, zm_ref, zx_ref):
        deg = d0_ref[...] + d1_ref[...]
        dis = jnp.where(deg > 0, lax.rsqrt(jnp.maximum(deg, 1.0)), 0.0)
        m = m_ref[...]
        dm = dis * m
        xv = x_ref[...]
        xv = jnp.where(jnp.isnan(xv), 0.0, xv)
        zm_ref[...] = xv * dm
        zx_ref[...] = jnp.concatenate([dis, dm], axis=1)

    return pl.pallas_call(
        zk,
        grid=(n_pad // BLK,),
        in_specs=[
            pl.BlockSpec((BLK, d), lambda i: (i, 0)),
            pl.BlockSpec((BLK, 1), lambda i: (i, 0)),
            pl.BlockSpec((BLK, 1), lambda i: (i, 0)),
            pl.BlockSpec((BLK, 1), lambda i: (i, 0)),
        ],
        out_specs=[
            pl.BlockSpec((BLK, d), lambda i: (i, 0)),
            pl.BlockSpec((BLK, 2), lambda i: (i, 0)),
        ],
        out_shape=[
            jax.ShapeDtypeStruct((n_pad, d), jnp.float32),
            jax.ShapeDtypeStruct((n_pad, 2), jnp.float32),
        ],
    )(x_pad, m_pad, deg0, deg1)


def _tc_final(acc0, acc1, a0_, a1_, c0_, c1_, deg0, deg1, W, b2, n_pad, d):

    def fk(b0_ref, b1_ref, a0_ref, a1_ref, c0_ref, c1_ref,
           d0_ref, d1_ref, w_ref, b_ref, o_ref):
        bsum = b0_ref[...] + b1_ref[...]
        asum = a0_ref[...] + a1_ref[...]
        csum = c0_ref[...] + c1_ref[...]
        deg = d0_ref[...] + d1_ref[...]
        dis = jnp.where(deg > 0, lax.rsqrt(jnp.maximum(deg, 1.0)), 0.0)
        nz = csum != 0.0
        scale = jnp.where(nz, dis * asum / jnp.where(nz, csum, 1.0), 0.0)
        ratio = bsum * scale
        out = lax.dot_general(ratio, w_ref[...], (((1,), (1,)), ((), ())),
                              preferred_element_type=jnp.float32)
        o_ref[...] = out + b_ref[...]

    col1 = pl.BlockSpec((BLK, 1), lambda i: (i, 0))
    return pl.pallas_call(
        fk,
        grid=(n_pad // BLK,),
        in_specs=[
            pl.BlockSpec((BLK, d), lambda i: (i, 0)),
            pl.BlockSpec((BLK, d), lambda i: (i, 0)),
            col1, col1, col1, col1, col1, col1,
            pl.BlockSpec((d, d), lambda i: (0, 0)),
            pl.BlockSpec((1, d), lambda i: (0, 0)),
        ],
        out_specs=pl.BlockSpec((BLK, d), lambda i: (i, 0)),
        out_shape=jax.ShapeDtypeStruct((n_pad, d), jnp.float32),
    )(acc0, acc1, a0_, a1_, c0_, c1_, deg0, deg1, W, b2)


def kernel(x, edge_index, train_mask, W, b):
    n, d = x.shape
    e = edge_index.shape[1]
    dt = DT
    n_pad = N_PAD

    nchunk = -(-e // (NW * K))
    nchunk += nchunk % 2
    e_pad = NW * nchunk * K
    trash = n_pad - 1

    pad_len = e_pad - e
    row = jnp.concatenate(
        [edge_index[0], jnp.full((pad_len,), trash, jnp.int32)])
    col = jnp.concatenate(
        [edge_index[1], jnp.full((pad_len,), trash, jnp.int32)])
    rc3 = (jnp.left_shift(row, IDXBITS) | col).reshape(NW, nchunk, K)

    x_pad = jnp.pad(x, ((0, n_pad - n), (0, 0)))
    m_pad = jnp.pad(train_mask, ((0, n_pad - n), (0, 0)))

    deg2 = _sc_degree(rc3, n_pad)
    deg0 = deg2[0].reshape(n_pad, 1)
    deg1 = deg2[1].reshape(n_pad, 1)

    zm, zx = _tc_build_z(x_pad, m_pad, deg0, deg1, n_pad, d)
    dis1 = zx[:, 0]
    dm1 = zx[:, 1]

    acc, ax, cx = _sc_spmm(zm, dis1, dm1, rc3, n_pad, d)

    out = _tc_final(acc[0], acc[1],
                    ax[0].reshape(n_pad, 1), ax[1].reshape(n_pad, 1),
                    cx[0].reshape(n_pad, 1), cx[1].reshape(n_pad, 1),
                    deg0, deg1, W, b.reshape(1, d), n_pad, d)
    return out[:n]


# speedup vs baseline: 17.9877x; 1.3781x over previous
import functools

import jax
import jax.numpy as jnp
from jax import lax
from jax.experimental import pallas as pl
from jax.experimental.pallas import tpu as pltpu
from jax.experimental.pallas import tpu_sc as plsc

NC = 2
NS = 16
NW = NC * NS
LANES = 16
K = 96
N_PAD = 10112
IDXBITS = 14
BLK = 128
DT = 136


def _mesh():
    return plsc.VectorSubcoreMesh(
        core_axis_name="c", subcore_axis_name="s",
        num_cores=NC, num_subcores=NS)


def _sc_params():
    return pltpu.CompilerParams(use_tc_tiling_on_sc=False)


def _sc_degree(row3, n_pad):
    nchunk = row3.shape[1]
    rps = n_pad // NS

    @functools.partial(
        pl.kernel,
        out_type=jax.ShapeDtypeStruct((NC, n_pad), jnp.float32),
        mesh=_mesh(),
        compiler_params=_sc_params(),
        scratch_types=[
            pltpu.VMEM((nchunk, K), jnp.int32),
            pltpu.VMEM((K,), jnp.float32),
            pltpu.VMEM_SHARED((n_pad,), jnp.float32),
        ],
    )
    def deg_kernel(row_hbm, ones_hbm, zeros_hbm, deg_hbm,
                   row_v, ones_v, deg_sp):
        c = lax.axis_index("c")
        s = lax.axis_index("s")
        t = c * NS + s

        pltpu.sync_copy(ones_hbm, ones_v)
        pltpu.sync_copy(zeros_hbm, deg_sp.at[pl.ds(s * rps, rps)])
        pltpu.sync_copy(row_hbm.at[t], row_v)
        plsc.subcore_barrier()

        def body(j, carry):
            pltpu.sync_copy(ones_v, deg_sp.at[row_v.at[j]], add=True)
            return carry
        lax.fori_loop(0, nchunk, body, 0)

        plsc.subcore_barrier()
        sl = pl.ds(s * rps, rps)
        pltpu.sync_copy(deg_sp.at[sl], deg_hbm.at[c].at[sl])

    return deg_kernel(row3, jnp.ones((K,), jnp.float32),
                      jnp.zeros((rps,), jnp.float32))


def _sc_spmm(z, dis1, dm1, row3, col3, n_pad, d):
    nchunk = row3.shape[1]
    rps = n_pad // NS

    @functools.partial(
        pl.kernel,
        out_type=[
            jax.ShapeDtypeStruct((NC, n_pad, d), jnp.float32),
            jax.ShapeDtypeStruct((NC, n_pad), jnp.float32),
            jax.ShapeDtypeStruct((NC, n_pad), jnp.float32),
        ],
        mesh=_mesh(),
        compiler_params=_sc_params(),
        scratch_types=[
            pltpu.VMEM((nchunk, K), jnp.int32),
            pltpu.VMEM((nchunk, K), jnp.int32),
            pltpu.VMEM((2, K, d), jnp.float32),
            pltpu.VMEM((2, K), jnp.float32),
            pltpu.VMEM((2, K), jnp.float32),
            pltpu.SemaphoreType.DMA((2,)),
            pltpu.SemaphoreType.DMA((2,)),
            pltpu.VMEM_SHARED((n_pad, d), jnp.float32),
            pltpu.VMEM_SHARED((n_pad,), jnp.float32),
            pltpu.VMEM_SHARED((n_pad,), jnp.float32),
        ],
    )
    def spmm_kernel(z_hbm, dis_hbm, dm_hbm, row_hbm, col_hbm,
                    zeros_hbm, zeros1_hbm,
                    acc_hbm, ax_hbm, cx_hbm,
                    row_v, col_v, gbuf, abuf, cbuf, gsem, ssem,
                    acc_sp, a_sp, c_sp):
        c = lax.axis_index("c")
        s = lax.axis_index("s")
        t = c * NS + s

        pltpu.sync_copy(zeros_hbm, acc_sp.at[pl.ds(s * rps, rps)])
        pltpu.sync_copy(zeros1_hbm, a_sp.at[pl.ds(s * rps, rps)])
        pltpu.sync_copy(zeros1_hbm, c_sp.at[pl.ds(s * rps, rps)])
        pltpu.sync_copy(row_hbm.at[t], row_v)
        pltpu.sync_copy(col_hbm.at[t], col_v)
        plsc.subcore_barrier()

        def start_gather(j, p):
            pltpu.async_copy(z_hbm.at[col_v.at[j]], gbuf.at[p], gsem.at[p])

        def wait_gather(j, p):
            pltpu.make_async_copy(z_hbm.at[col_v.at[j]], gbuf.at[p], gsem.at[p]).wait()

        def start_scatter(j, p):
            pltpu.async_copy(gbuf.at[p], acc_sp.at[row_v.at[j]], ssem.at[p], add=True)

        def wait_scatter(j, p):
            pltpu.make_async_copy(gbuf.at[p], acc_sp.at[row_v.at[j]], ssem.at[p]).wait()

        def small_chunk(j, p):
            pltpu.sync_copy(dis_hbm.at[col_v.at[j]], abuf.at[p])
            pltpu.sync_copy(dm_hbm.at[col_v.at[j]], cbuf.at[p])
            pltpu.sync_copy(abuf.at[p], a_sp.at[row_v.at[j]], add=True)
            pltpu.sync_copy(cbuf.at[p], c_sp.at[row_v.at[j]], add=True)

        start_gather(0, 0)

        def body(i, carry):
            j0 = 2 * i
            wait_gather(j0, 0)
            start_scatter(j0, 0)
            @pl.when(j0 > 0)
            def _():
                wait_scatter(j0 - 1, 1)
            start_gather(j0 + 1, 1)
            small_chunk(j0, 0)
            wait_gather(j0 + 1, 1)
            start_scatter(j0 + 1, 1)
            wait_scatter(j0, 0)
            @pl.when(j0 + 2 < nchunk)
            def _():
                start_gather(j0 + 2, 0)
            small_chunk(j0 + 1, 1)
            return carry
        lax.fori_loop(0, nchunk // 2, body, 0)
        wait_scatter(nchunk - 1, 1)

        plsc.subcore_barrier()
        sl = pl.ds(s * rps, rps)
        pltpu.sync_copy(acc_sp.at[sl], acc_hbm.at[c].at[sl])
        pltpu.sync_copy(a_sp.at[sl], ax_hbm.at[c].at[sl])
        pltpu.sync_copy(c_sp.at[sl], cx_hbm.at[c].at[sl])

    return spmm_kernel(z, dis1, dm1, row3, col3,
                       jnp.zeros((rps, d), jnp.float32),
                       jnp.zeros((rps,), jnp.float32))


def _tc_build_z(x_pad, m_pad, deg0, deg1, n_pad, d):
    def zk(x_ref, m_ref, d0_ref, d1_ref, zm_ref, zx_ref):
        deg = d0_ref[...] + d1_ref[...]
        dis = jnp.where(deg > 0, lax.rsqrt(jnp.maximum(deg, 1.0)), 0.0)
        m = m_ref[...]
        dm = dis * m
        xv = x_ref[...]
        xv = jnp.where(jnp.isnan(xv), 0.0, xv)
        zm_ref[...] = xv * dm
        zx_ref[...] = jnp.concatenate([dis, dm], axis=1)

    return pl.pallas_call(
        zk,
        grid=(n_pad // BLK,),
        in_specs=[
            pl.BlockSpec((BLK, d), lambda i: (i, 0)),
            pl.BlockSpec((BLK, 1), lambda i: (i, 0)),
            pl.BlockSpec((BLK, 1), lambda i: (i, 0)),
            pl.BlockSpec((BLK, 1), lambda i: (i, 0)),
        ],
        out_specs=[
            pl.BlockSpec((BLK, d), lambda i: (i, 0)),
            pl.BlockSpec((BLK, 2), lambda i: (i, 0)),
        ],
        out_shape=[
            jax.ShapeDtypeStruct((n_pad, d), jnp.float32),
            jax.ShapeDtypeStruct((n_pad, 2), jnp.float32),
        ],
    )(x_pad, m_pad, deg0, deg1)


def _tc_final(acc0, acc1, a0_, a1_, c0_, c1_, deg0, deg1, W, b2, n_pad, d):

    def fk(b0_ref, b1_ref, a0_ref, a1_ref, c0_ref, c1_ref,
           d0_ref, d1_ref, w_ref, b_ref, o_ref):
        bsum = b0_ref[...] + b1_ref[...]
        asum = a0_ref[...] + a1_ref[...]
        csum = c0_ref[...] + c1_ref[...]
        deg = d0_ref[...] + d1_ref[...]
        dis = jnp.where(deg > 0, lax.rsqrt(jnp.maximum(deg, 1.0)), 0.0)
        nz = csum != 0.0
        scale = jnp.where(nz, dis * asum / jnp.where(nz, csum, 1.0), 0.0)
        ratio = bsum * scale
        out = lax.dot_general(ratio, w_ref[...], (((1,), (1,)), ((), ())),
                              preferred_element_type=jnp.float32)
        o_ref[...] = out + b_ref[...]

    col1 = pl.BlockSpec((BLK, 1), lambda i: (i, 0))
    return pl.pallas_call(
        fk,
        grid=(n_pad // BLK,),
        in_specs=[
            pl.BlockSpec((BLK, d), lambda i: (i, 0)),
            pl.BlockSpec((BLK, d), lambda i: (i, 0)),
            col1, col1, col1, col1, col1, col1,
            pl.BlockSpec((d, d), lambda i: (0, 0)),
            pl.BlockSpec((1, d), lambda i: (0, 0)),
        ],
        out_specs=pl.BlockSpec((BLK, d), lambda i: (i, 0)),
        out_shape=jax.ShapeDtypeStruct((n_pad, d), jnp.float32),
    )(acc0, acc1, a0_, a1_, c0_, c1_, deg0, deg1, W, b2)


def kernel(x, edge_index, train_mask, W, b):
    n, d = x.shape
    e = edge_index.shape[1]
    dt = DT
    n_pad = N_PAD

    nchunk = -(-e // (NW * K))
    nchunk += nchunk % 2
    e_pad = NW * nchunk * K
    trash = n_pad - 1

    pad_len = e_pad - e
    row = jnp.concatenate(
        [edge_index[0], jnp.full((pad_len,), trash, jnp.int32)])
    col = jnp.concatenate(
        [edge_index[1], jnp.full((pad_len,), trash, jnp.int32)])
    row3 = row.reshape(NW, nchunk, K)
    col3 = col.reshape(NW, nchunk, K)

    x_pad = jnp.pad(x, ((0, n_pad - n), (0, 0)))
    m_pad = jnp.pad(train_mask, ((0, n_pad - n), (0, 0)))

    deg2 = _sc_degree(row3, n_pad)
    deg0 = deg2[0].reshape(n_pad, 1)
    deg1 = deg2[1].reshape(n_pad, 1)

    zm, zx = _tc_build_z(x_pad, m_pad, deg0, deg1, n_pad, d)
    dis1 = zx[:, 0]
    dm1 = zx[:, 1]

    acc, ax, cx = _sc_spmm(zm, dis1, dm1, row3, col3, n_pad, d)

    out = _tc_final(acc[0], acc[1],
                    ax[0].reshape(n_pad, 1), ax[1].reshape(n_pad, 1),
                    cx[0].reshape(n_pad, 1), cx[1].reshape(n_pad, 1),
                    deg0, deg1, W, b.reshape(1, d), n_pad, d)
    return out[:n]


# speedup vs baseline: 18.1788x; 1.0106x over previous
import functools

import jax
import jax.numpy as jnp
from jax import lax
from jax.experimental import pallas as pl
from jax.experimental.pallas import tpu as pltpu
from jax.experimental.pallas import tpu_sc as plsc

NC = 2
NS = 16
NW = NC * NS
LANES = 16
K = 96
N_PAD = 10112
IDXBITS = 14
BLK = 128
DT = 136


def _mesh():
    return plsc.VectorSubcoreMesh(
        core_axis_name="c", subcore_axis_name="s",
        num_cores=NC, num_subcores=NS)


def _sc_params():
    return pltpu.CompilerParams(use_tc_tiling_on_sc=False)


def _sc_degree(row3, n_pad):
    nchunk = row3.shape[1]
    rps = n_pad // NS

    @functools.partial(
        pl.kernel,
        out_type=jax.ShapeDtypeStruct((NC, n_pad), jnp.float32),
        mesh=_mesh(),
        compiler_params=_sc_params(),
        scratch_types=[
            pltpu.VMEM((nchunk, K), jnp.int32),
            pltpu.VMEM((K,), jnp.float32),
            pltpu.VMEM_SHARED((n_pad,), jnp.float32),
        ],
    )
    def deg_kernel(row_hbm, ones_hbm, zeros_hbm, deg_hbm,
                   row_v, ones_v, deg_sp):
        c = lax.axis_index("c")
        s = lax.axis_index("s")
        t = c * NS + s

        pltpu.sync_copy(ones_hbm, ones_v)
        pltpu.sync_copy(zeros_hbm, deg_sp.at[pl.ds(s * rps, rps)])
        pltpu.sync_copy(row_hbm.at[t], row_v)
        plsc.subcore_barrier()

        def body(j, carry):
            pltpu.sync_copy(ones_v, deg_sp.at[row_v.at[j]], add=True)
            return carry
        lax.fori_loop(0, nchunk, body, 0)

        plsc.subcore_barrier()
        sl = pl.ds(s * rps, rps)
        pltpu.sync_copy(deg_sp.at[sl], deg_hbm.at[c].at[sl])

    return deg_kernel(row3, jnp.ones((K,), jnp.float32),
                      jnp.zeros((rps,), jnp.float32))


def _sc_spmm(z, dis1, dm1, row3, col3, n_pad, d):
    nchunk = row3.shape[1]
    rps = n_pad // NS

    @functools.partial(
        pl.kernel,
        out_type=[
            jax.ShapeDtypeStruct((NC, n_pad, d), jnp.float32),
            jax.ShapeDtypeStruct((NC, n_pad), jnp.float32),
            jax.ShapeDtypeStruct((NC, n_pad), jnp.float32),
        ],
        mesh=_mesh(),
        compiler_params=_sc_params(),
        scratch_types=[
            pltpu.VMEM((nchunk, K), jnp.int32),
            pltpu.VMEM((nchunk, K), jnp.int32),
            pltpu.VMEM((2, K, d), jnp.float32),
            pltpu.VMEM((2, K), jnp.float32),
            pltpu.VMEM((2, K), jnp.float32),
            pltpu.SemaphoreType.DMA((2,)),
            pltpu.SemaphoreType.DMA((2,)),
            pltpu.VMEM_SHARED((n_pad, d), jnp.float32),
            pltpu.VMEM_SHARED((n_pad,), jnp.float32),
            pltpu.VMEM_SHARED((n_pad,), jnp.float32),
        ],
    )
    def spmm_kernel(z_hbm, dis_hbm, dm_hbm, row_hbm, col_hbm,
                    zeros_hbm, zeros1_hbm,
                    acc_hbm, ax_hbm, cx_hbm,
                    row_v, col_v, gbuf, abuf, cbuf, gsem, ssem,
                    acc_sp, a_sp, c_sp):
        c = lax.axis_index("c")
        s = lax.axis_index("s")
        t = c * NS + s

        pltpu.sync_copy(zeros_hbm, acc_sp.at[pl.ds(s * rps, rps)])
        pltpu.sync_copy(zeros1_hbm, a_sp.at[pl.ds(s * rps, rps)])
        pltpu.sync_copy(zeros1_hbm, c_sp.at[pl.ds(s * rps, rps)])
        pltpu.sync_copy(row_hbm.at[t], row_v)
        pltpu.sync_copy(col_hbm.at[t], col_v)
        plsc.subcore_barrier()

        def start_gather(j, p):
            pltpu.async_copy(z_hbm.at[col_v.at[j]], gbuf.at[p], gsem.at[p])
            pltpu.async_copy(dis_hbm.at[col_v.at[j]], abuf.at[p], gsem.at[p])
            pltpu.async_copy(dm_hbm.at[col_v.at[j]], cbuf.at[p], gsem.at[p])

        def wait_gather(j, p):
            pltpu.make_async_copy(z_hbm.at[col_v.at[j]], gbuf.at[p], gsem.at[p]).wait()
            pltpu.make_async_copy(dis_hbm.at[col_v.at[j]], abuf.at[p], gsem.at[p]).wait()
            pltpu.make_async_copy(dm_hbm.at[col_v.at[j]], cbuf.at[p], gsem.at[p]).wait()

        def start_scatter(j, p):
            pltpu.async_copy(gbuf.at[p], acc_sp.at[row_v.at[j]], ssem.at[p], add=True)
            pltpu.async_copy(abuf.at[p], a_sp.at[row_v.at[j]], ssem.at[p], add=True)
            pltpu.async_copy(cbuf.at[p], c_sp.at[row_v.at[j]], ssem.at[p], add=True)

        def wait_scatter(j, p):
            pltpu.make_async_copy(gbuf.at[p], acc_sp.at[row_v.at[j]], ssem.at[p]).wait()
            pltpu.make_async_copy(abuf.at[p], a_sp.at[row_v.at[j]], ssem.at[p]).wait()
            pltpu.make_async_copy(cbuf.at[p], c_sp.at[row_v.at[j]], ssem.at[p]).wait()

        start_gather(0, 0)

        def body(i, carry):
            j0 = 2 * i
            wait_gather(j0, 0)
            start_scatter(j0, 0)
            @pl.when(j0 > 0)
            def _():
                wait_scatter(j0 - 1, 1)
            start_gather(j0 + 1, 1)
            wait_gather(j0 + 1, 1)
            start_scatter(j0 + 1, 1)
            wait_scatter(j0, 0)
            @pl.when(j0 + 2 < nchunk)
            def _():
                start_gather(j0 + 2, 0)
            return carry
        lax.fori_loop(0, nchunk // 2, body, 0)
        wait_scatter(nchunk - 1, 1)

        plsc.subcore_barrier()
        sl = pl.ds(s * rps, rps)
        pltpu.sync_copy(acc_sp.at[sl], acc_hbm.at[c].at[sl])
        pltpu.sync_copy(a_sp.at[sl], ax_hbm.at[c].at[sl])
        pltpu.sync_copy(c_sp.at[sl], cx_hbm.at[c].at[sl])

    return spmm_kernel(z, dis1, dm1, row3, col3,
                       jnp.zeros((rps, d), jnp.float32),
                       jnp.zeros((rps,), jnp.float32))


def _tc_build_z(x_pad, m_pad, deg0, deg1, n_pad, d):
    def zk(x_ref, m_ref, d0_ref, d1_ref, zm_ref, zx_ref):
        deg = d0_ref[...] + d1_ref[...]
        dis = jnp.where(deg > 0, lax.rsqrt(jnp.maximum(deg, 1.0)), 0.0)
        m = m_ref[...]
        dm = dis * m
        xv = x_ref[...]
        xv = jnp.where(jnp.isnan(xv), 0.0, xv)
        zm_ref[...] = xv * dm
        zx_ref[...] = jnp.concatenate([dis, dm], axis=1)

    return pl.pallas_call(
        zk,
        grid=(n_pad // BLK,),
        in_specs=[
            pl.BlockSpec((BLK, d), lambda i: (i, 0)),
            pl.BlockSpec((BLK, 1), lambda i: (i, 0)),
            pl.BlockSpec((BLK, 1), lambda i: (i, 0)),
            pl.BlockSpec((BLK, 1), lambda i: (i, 0)),
        ],
        out_specs=[
            pl.BlockSpec((BLK, d), lambda i: (i, 0)),
            pl.BlockSpec((BLK, 2), lambda i: (i, 0)),
        ],
        out_shape=[
            jax.ShapeDtypeStruct((n_pad, d), jnp.float32),
            jax.ShapeDtypeStruct((n_pad, 2), jnp.float32),
        ],
    )(x_pad, m_pad, deg0, deg1)


def _tc_final(acc0, acc1, a0_, a1_, c0_, c1_, deg0, deg1, W, b2, n_pad, d):

    def fk(b0_ref, b1_ref, a0_ref, a1_ref, c0_ref, c1_ref,
           d0_ref, d1_ref, w_ref, b_ref, o_ref):
        bsum = b0_ref[...] + b1_ref[...]
        asum = a0_ref[...] + a1_ref[...]
        csum = c0_ref[...] + c1_ref[...]
        deg = d0_ref[...] + d1_ref[...]
        dis = jnp.where(deg > 0, lax.rsqrt(jnp.maximum(deg, 1.0)), 0.0)
        nz = csum != 0.0
        scale = jnp.where(nz, dis * asum / jnp.where(nz, csum, 1.0), 0.0)
        ratio = bsum * scale
        out = lax.dot_general(ratio, w_ref[...], (((1,), (1,)), ((), ())),
                              preferred_element_type=jnp.float32)
        o_ref[...] = out + b_ref[...]

    col1 = pl.BlockSpec((BLK, 1), lambda i: (i, 0))
    return pl.pallas_call(
        fk,
        grid=(n_pad // BLK,),
        in_specs=[
            pl.BlockSpec((BLK, d), lambda i: (i, 0)),
            pl.BlockSpec((BLK, d), lambda i: (i, 0)),
            col1, col1, col1, col1, col1, col1,
            pl.BlockSpec((d, d), lambda i: (0, 0)),
            pl.BlockSpec((1, d), lambda i: (0, 0)),
        ],
        out_specs=pl.BlockSpec((BLK, d), lambda i: (i, 0)),
        out_shape=jax.ShapeDtypeStruct((n_pad, d), jnp.float32),
    )(acc0, acc1, a0_, a1_, c0_, c1_, deg0, deg1, W, b2)


def kernel(x, edge_index, train_mask, W, b):
    n, d = x.shape
    e = edge_index.shape[1]
    dt = DT
    n_pad = N_PAD

    nchunk = -(-e // (NW * K))
    nchunk += nchunk % 2
    e_pad = NW * nchunk * K
    trash = n_pad - 1

    pad_len = e_pad - e
    row = jnp.concatenate(
        [edge_index[0], jnp.full((pad_len,), trash, jnp.int32)])
    col = jnp.concatenate(
        [edge_index[1], jnp.full((pad_len,), trash, jnp.int32)])
    row3 = row.reshape(NW, nchunk, K)
    col3 = col.reshape(NW, nchunk, K)

    x_pad = jnp.pad(x, ((0, n_pad - n), (0, 0)))
    m_pad = jnp.pad(train_mask, ((0, n_pad - n), (0, 0)))

    deg2 = _sc_degree(row3, n_pad)
    deg0 = deg2[0].reshape(n_pad, 1)
    deg1 = deg2[1].reshape(n_pad, 1)

    zm, zx = _tc_build_z(x_pad, m_pad, deg0, deg1, n_pad, d)
    dis1 = zx[:, 0]
    dm1 = zx[:, 1]

    acc, ax, cx = _sc_spmm(zm, dis1, dm1, row3, col3, n_pad, d)

    out = _tc_final(acc[0], acc[1],
                    ax[0].reshape(n_pad, 1), ax[1].reshape(n_pad, 1),
                    cx[0].reshape(n_pad, 1), cx[1].reshape(n_pad, 1),
                    deg0, deg1, W, b.reshape(1, d), n_pad, d)
    return out[:n]


# speedup vs baseline: 27.3077x; 1.5022x over previous
import functools

import jax
import jax.numpy as jnp
from jax import lax
from jax.experimental import pallas as pl
from jax.experimental.pallas import tpu as pltpu
from jax.experimental.pallas import tpu_sc as plsc

NC = 2
NS = 16
NW = NC * NS
K = 88
N_PAD = 10112
BLK = 128
DT = 136


def _mesh():
    return plsc.VectorSubcoreMesh(
        core_axis_name="c", subcore_axis_name="s",
        num_cores=NC, num_subcores=NS)


def _sc_params():
    return pltpu.CompilerParams(use_tc_tiling_on_sc=False)


def _sc_degree(row3, n_pad):
    nchunk = row3.shape[1]
    rps = n_pad // NS

    @functools.partial(
        pl.kernel,
        out_type=jax.ShapeDtypeStruct((NC, n_pad), jnp.float32),
        mesh=_mesh(),
        compiler_params=_sc_params(),
        scratch_types=[
            pltpu.VMEM((nchunk, K), jnp.int32),
            pltpu.VMEM((K,), jnp.float32),
            pltpu.VMEM_SHARED((n_pad,), jnp.float32),
        ],
    )
    def deg_kernel(row_hbm, ones_hbm, zeros_hbm, deg_hbm,
                   row_v, ones_v, deg_sp):
        c = lax.axis_index("c")
        s = lax.axis_index("s")
        t = c * NS + s

        pltpu.sync_copy(ones_hbm, ones_v)
        pltpu.sync_copy(zeros_hbm, deg_sp.at[pl.ds(s * rps, rps)])
        pltpu.sync_copy(row_hbm.at[t], row_v)
        plsc.subcore_barrier()

        def body(j, carry):
            pltpu.sync_copy(ones_v, deg_sp.at[row_v.at[j]], add=True)
            return carry
        lax.fori_loop(0, nchunk, body, 0)

        plsc.subcore_barrier()
        sl = pl.ds(s * rps, rps)
        pltpu.sync_copy(deg_sp.at[sl], deg_hbm.at[c].at[sl])

    return deg_kernel(row3, jnp.ones((K,), jnp.float32),
                      jnp.zeros((rps,), jnp.float32))


def _sc_spmm(z, row3, col3, n_pad, dt):
    nchunk = row3.shape[1]
    rps = n_pad // NS

    @functools.partial(
        pl.kernel,
        out_type=jax.ShapeDtypeStruct((NC, n_pad, dt), jnp.float32),
        mesh=_mesh(),
        compiler_params=_sc_params(),
        scratch_types=[
            pltpu.VMEM((nchunk, K), jnp.int32),
            pltpu.VMEM((nchunk, K), jnp.int32),
            pltpu.VMEM((2, K, dt), jnp.float32),
            pltpu.SemaphoreType.DMA((2,)),
            pltpu.SemaphoreType.DMA((2,)),
            pltpu.VMEM_SHARED((n_pad, dt), jnp.float32),
        ],
    )
    def spmm_kernel(z_hbm, row_hbm, col_hbm, zeros_hbm, acc_hbm,
                    row_v, col_v, gbuf, gsem, ssem, acc_sp):
        c = lax.axis_index("c")
        s = lax.axis_index("s")
        t = c * NS + s

        pltpu.sync_copy(zeros_hbm, acc_sp.at[pl.ds(s * rps, rps)])
        pltpu.sync_copy(row_hbm.at[t], row_v)
        pltpu.sync_copy(col_hbm.at[t], col_v)
        plsc.subcore_barrier()

        def start_gather(j, p):
            pltpu.async_copy(z_hbm.at[col_v.at[j]], gbuf.at[p], gsem.at[p])

        def wait_gather(j, p):
            pltpu.make_async_copy(z_hbm.at[col_v.at[j]], gbuf.at[p],
                                  gsem.at[p]).wait()

        def start_scatter(j, p):
            pltpu.async_copy(gbuf.at[p], acc_sp.at[row_v.at[j]],
                             ssem.at[p], add=True)

        def wait_scatter(j, p):
            pltpu.make_async_copy(gbuf.at[p], acc_sp.at[row_v.at[j]],
                                  ssem.at[p]).wait()

        start_gather(0, 0)

        def body(i, carry):
            j0 = 2 * i
            wait_gather(j0, 0)
            start_scatter(j0, 0)
            @pl.when(j0 > 0)
            def _():
                wait_scatter(j0 - 1, 1)
            start_gather(j0 + 1, 1)
            wait_gather(j0 + 1, 1)
            start_scatter(j0 + 1, 1)
            wait_scatter(j0, 0)
            @pl.when(j0 + 2 < nchunk)
            def _():
                start_gather(j0 + 2, 0)
            return carry
        lax.fori_loop(0, nchunk // 2, body, 0)
        wait_scatter(nchunk - 1, 1)

        plsc.subcore_barrier()
        sl = pl.ds(s * rps, rps)
        pltpu.sync_copy(acc_sp.at[sl], acc_hbm.at[c].at[sl])

    return spmm_kernel(z, row3, col3, jnp.zeros((rps, dt), jnp.float32))


def _tc_build_z(x, m, deg0, deg1, n_pad, d):
    def zk(x_ref, m_ref, d0_ref, d1_ref, z_ref):
        deg = d0_ref[...] + d1_ref[...]
        dis = jnp.where(deg > 0, lax.rsqrt(jnp.maximum(deg, 1.0)), 0.0)
        mv = m_ref[...]
        dm = dis * mv
        xv = x_ref[...]
        xv = jnp.where(jnp.isnan(xv), 0.0, xv)
        z_ref[...] = jnp.concatenate(
            [xv * dm, dis, dm, jnp.zeros((BLK, DT - d - 2), jnp.float32)],
            axis=1)

    return pl.pallas_call(
        zk,
        grid=(n_pad // BLK,),
        in_specs=[
            pl.BlockSpec((BLK, d), lambda i: (i, 0)),
            pl.BlockSpec((BLK, 1), lambda i: (i, 0)),
            pl.BlockSpec((BLK, 1), lambda i: (i, 0)),
            pl.BlockSpec((BLK, 1), lambda i: (i, 0)),
        ],
        out_specs=pl.BlockSpec((BLK, DT), lambda i: (i, 0)),
        out_shape=jax.ShapeDtypeStruct((n_pad, DT), jnp.float32),
    )(x, m, deg0, deg1)


def _tc_final(acc, deg0, deg1, W, b2, n, n_pad, d, dt):

    def fk(a_ref, d0_ref, d1_ref, w_ref, b_ref, o_ref):
        a0 = a_ref[0]
        a1 = a_ref[1]
        bsum = a0[:, :d] + a1[:, :d]
        asum = a0[:, d:d + 1] + a1[:, d:d + 1]
        csum = a0[:, d + 1:d + 2] + a1[:, d + 1:d + 2]
        deg = d0_ref[...] + d1_ref[...]
        dis = jnp.where(deg > 0, lax.rsqrt(jnp.maximum(deg, 1.0)), 0.0)
        nz = csum != 0.0
        scale = jnp.where(nz, dis * asum / jnp.where(nz, csum, 1.0), 0.0)
        ratio = bsum * scale
        out = lax.dot_general(ratio, w_ref[...], (((1,), (1,)), ((), ())),
                              preferred_element_type=jnp.float32)
        o_ref[...] = out + b_ref[...]

    return pl.pallas_call(
        fk,
        grid=(n_pad // BLK,),
        in_specs=[
            pl.BlockSpec((2, BLK, dt), lambda i: (0, i, 0)),
            pl.BlockSpec((BLK, 1), lambda i: (i, 0)),
            pl.BlockSpec((BLK, 1), lambda i: (i, 0)),
            pl.BlockSpec((d, d), lambda i: (0, 0)),
            pl.BlockSpec((1, d), lambda i: (0, 0)),
        ],
        out_specs=pl.BlockSpec((BLK, d), lambda i: (i, 0)),
        out_shape=jax.ShapeDtypeStruct((n, d), jnp.float32),
    )(acc, deg0, deg1, W, b2)


def kernel(x, edge_index, train_mask, W, b):
    n, d = x.shape
    e = edge_index.shape[1]
    n_pad = N_PAD

    nchunk = -(-e // (NW * K))
    nchunk += nchunk % 2
    e_pad = NW * nchunk * K
    trash = n_pad - 1

    pad_len = e_pad - e
    row = jnp.concatenate(
        [edge_index[0], jnp.full((pad_len,), trash, jnp.int32)])
    col = jnp.concatenate(
        [edge_index[1], jnp.full((pad_len,), trash, jnp.int32)])
    row3 = row.reshape(NW, nchunk, K)
    col3 = col.reshape(NW, nchunk, K)

    deg2 = _sc_degree(row3, n_pad)
    deg0 = deg2[0].reshape(n_pad, 1)
    deg1 = deg2[1].reshape(n_pad, 1)

    z = _tc_build_z(x, train_mask, deg0, deg1, n_pad, d)

    acc = _sc_spmm(z, row3, col3, n_pad, DT)

    return _tc_final(acc, deg0, deg1, W, b.reshape(1, d), n, n_pad, d, DT)


# speedup vs baseline: 30.6937x; 1.1240x over previous
import functools

import jax
import jax.numpy as jnp
from jax import lax
from jax.experimental import pallas as pl
from jax.experimental.pallas import tpu as pltpu
from jax.experimental.pallas import tpu_sc as plsc

NC = 2
NS = 16
NW = NC * NS
K = 88
GRP = 6
N_PAD = 10112
BLK = 128
DT = 136


def _mesh():
    return plsc.VectorSubcoreMesh(
        core_axis_name="c", subcore_axis_name="s",
        num_cores=NC, num_subcores=NS)


def _sc_params():
    return pltpu.CompilerParams(use_tc_tiling_on_sc=False)


def _sc_degree(row3, n_pad):
    nchunk = row3.shape[1]
    rps = n_pad // NS

    @functools.partial(
        pl.kernel,
        out_type=jax.ShapeDtypeStruct((NC, n_pad), jnp.float32),
        mesh=_mesh(),
        compiler_params=_sc_params(),
        scratch_types=[
            pltpu.VMEM((nchunk, K), jnp.int32),
            pltpu.VMEM((K,), jnp.float32),
            pltpu.SemaphoreType.DMA,
            pltpu.VMEM_SHARED((n_pad,), jnp.float32),
        ],
    )
    def deg_kernel(row_hbm, ones_hbm, zeros_hbm, deg_hbm,
                   row_v, ones_v, dsem, deg_sp):
        c = lax.axis_index("c")
        s = lax.axis_index("s")
        t = c * NS + s

        pltpu.sync_copy(ones_hbm, ones_v)
        pltpu.sync_copy(zeros_hbm, deg_sp.at[pl.ds(s * rps, rps)])
        pltpu.sync_copy(row_hbm.at[t], row_v)
        plsc.subcore_barrier()

        def body(i, carry):
            j0 = i * GRP
            for u in range(GRP):
                pltpu.async_copy(ones_v, deg_sp.at[row_v.at[j0 + u]], dsem,
                                 add=True)
            for u in range(GRP):
                pltpu.make_async_copy(ones_v, deg_sp.at[row_v.at[j0 + u]],
                                      dsem).wait()
            return carry
        lax.fori_loop(0, nchunk // GRP, body, 0)

        plsc.subcore_barrier()
        sl = pl.ds(s * rps, rps)
        pltpu.sync_copy(deg_sp.at[sl], deg_hbm.at[c].at[sl])

    return deg_kernel(row3, jnp.ones((K,), jnp.float32),
                      jnp.zeros((rps,), jnp.float32))


def _sc_spmm(z, row3, col3, n_pad, dt):
    nchunk = row3.shape[1]
    rps = n_pad // NS

    @functools.partial(
        pl.kernel,
        out_type=jax.ShapeDtypeStruct((NC, n_pad, dt), jnp.float32),
        mesh=_mesh(),
        compiler_params=_sc_params(),
        scratch_types=[
            pltpu.VMEM((nchunk, K), jnp.int32),
            pltpu.VMEM((nchunk, K), jnp.int32),
            pltpu.VMEM((2, K, dt), jnp.float32),
            pltpu.SemaphoreType.DMA((2,)),
            pltpu.SemaphoreType.DMA((2,)),
            pltpu.VMEM_SHARED((n_pad, dt), jnp.float32),
        ],
    )
    def spmm_kernel(z_hbm, row_hbm, col_hbm, zeros_hbm, acc_hbm,
                    row_v, col_v, gbuf, gsem, ssem, acc_sp):
        c = lax.axis_index("c")
        s = lax.axis_index("s")
        t = c * NS + s

        pltpu.sync_copy(zeros_hbm, acc_sp.at[pl.ds(s * rps, rps)])
        pltpu.sync_copy(row_hbm.at[t], row_v)
        pltpu.sync_copy(col_hbm.at[t], col_v)
        plsc.subcore_barrier()

        def start_gather(j, p):
            pltpu.async_copy(z_hbm.at[col_v.at[j]], gbuf.at[p], gsem.at[p])

        def wait_gather(j, p):
            pltpu.make_async_copy(z_hbm.at[col_v.at[j]], gbuf.at[p],
                                  gsem.at[p]).wait()

        def start_scatter(j, p):
            pltpu.async_copy(gbuf.at[p], acc_sp.at[row_v.at[j]],
                             ssem.at[p], add=True)

        def wait_scatter(j, p):
            pltpu.make_async_copy(gbuf.at[p], acc_sp.at[row_v.at[j]],
                                  ssem.at[p]).wait()

        start_gather(0, 0)

        def body(i, carry):
            j0 = 2 * i
            wait_gather(j0, 0)
            start_scatter(j0, 0)
            @pl.when(j0 > 0)
            def _():
                wait_scatter(j0 - 1, 1)
            start_gather(j0 + 1, 1)
            wait_gather(j0 + 1, 1)
            start_scatter(j0 + 1, 1)
            wait_scatter(j0, 0)
            @pl.when(j0 + 2 < nchunk)
            def _():
                start_gather(j0 + 2, 0)
            return carry
        lax.fori_loop(0, nchunk // 2, body, 0)
        wait_scatter(nchunk - 1, 1)

        plsc.subcore_barrier()
        sl = pl.ds(s * rps, rps)
        pltpu.sync_copy(acc_sp.at[sl], acc_hbm.at[c].at[sl])

    return spmm_kernel(z, row3, col3, jnp.zeros((rps, dt), jnp.float32))


def _tc_build_z(x, m, deg0, deg1, n_pad, d):
    def zk(x_ref, m_ref, d0_ref, d1_ref, z_ref):
        deg = d0_ref[...] + d1_ref[...]
        dis = jnp.where(deg > 0, lax.rsqrt(jnp.maximum(deg, 1.0)), 0.0)
        mv = m_ref[...]
        dm = dis * mv
        xv = x_ref[...]
        xv = jnp.where(jnp.isnan(xv), 0.0, xv)
        z_ref[...] = jnp.concatenate(
            [xv * dm, dis, dm, jnp.zeros((BLK, DT - d - 2), jnp.float32)],
            axis=1)

    return pl.pallas_call(
        zk,
        grid=(n_pad // BLK,),
        in_specs=[
            pl.BlockSpec((BLK, d), lambda i: (i, 0)),
            pl.BlockSpec((BLK, 1), lambda i: (i, 0)),
            pl.BlockSpec((BLK, 1), lambda i: (i, 0)),
            pl.BlockSpec((BLK, 1), lambda i: (i, 0)),
        ],
        out_specs=pl.BlockSpec((BLK, DT), lambda i: (i, 0)),
        out_shape=jax.ShapeDtypeStruct((n_pad, DT), jnp.float32),
    )(x, m, deg0, deg1)


def _tc_final(acc, deg0, deg1, W, b2, n, n_pad, d, dt):

    def fk(a_ref, d0_ref, d1_ref, w_ref, b_ref, o_ref):
        a0 = a_ref[0]
        a1 = a_ref[1]
        bsum = a0[:, :d] + a1[:, :d]
        asum = a0[:, d:d + 1] + a1[:, d:d + 1]
        csum = a0[:, d + 1:d + 2] + a1[:, d + 1:d + 2]
        deg = d0_ref[...] + d1_ref[...]
        dis = jnp.where(deg > 0, lax.rsqrt(jnp.maximum(deg, 1.0)), 0.0)
        nz = csum != 0.0
        scale = jnp.where(nz, dis * asum / jnp.where(nz, csum, 1.0), 0.0)
        ratio = bsum * scale
        out = lax.dot_general(ratio, w_ref[...], (((1,), (1,)), ((), ())),
                              preferred_element_type=jnp.float32)
        o_ref[...] = out + b_ref[...]

    return pl.pallas_call(
        fk,
        grid=(n_pad // BLK,),
        in_specs=[
            pl.BlockSpec((2, BLK, dt), lambda i: (0, i, 0)),
            pl.BlockSpec((BLK, 1), lambda i: (i, 0)),
            pl.BlockSpec((BLK, 1), lambda i: (i, 0)),
            pl.BlockSpec((d, d), lambda i: (0, 0)),
            pl.BlockSpec((1, d), lambda i: (0, 0)),
        ],
        out_specs=pl.BlockSpec((BLK, d), lambda i: (i, 0)),
        out_shape=jax.ShapeDtypeStruct((n, d), jnp.float32),
    )(acc, deg0, deg1, W, b2)


def kernel(x, edge_index, train_mask, W, b):
    n, d = x.shape
    e = edge_index.shape[1]
    n_pad = N_PAD

    nchunk = -(-e // (NW * K))
    nchunk = -(-nchunk // GRP) * GRP
    e_pad = NW * nchunk * K

    pad_len = e_pad - e
    pad_iota = jnp.arange(pad_len, dtype=jnp.int32)
    row = jnp.concatenate(
        [edge_index[0], n + pad_iota % (n_pad - n)])
    col = jnp.concatenate(
        [edge_index[1], pad_iota % n])
    row3 = row.reshape(NW, nchunk, K)
    col3 = col.reshape(NW, nchunk, K)

    deg2 = _sc_degree(row3, n_pad)
    deg0 = deg2[0].reshape(n_pad, 1)
    deg1 = deg2[1].reshape(n_pad, 1)

    z = _tc_build_z(x, train_mask, deg0, deg1, n_pad, d)

    acc = _sc_spmm(z, row3, col3, n_pad, DT)

    return _tc_final(acc, deg0, deg1, W, b.reshape(1, d), n, n_pad, d, DT)


# speedup vs baseline: 30.9855x; 1.0095x over previous
import functools

import jax
import jax.numpy as jnp
from jax import lax
from jax.experimental import pallas as pl
from jax.experimental.pallas import tpu as pltpu
from jax.experimental.pallas import tpu_sc as plsc

NC = 2
NS = 16
NW = NC * NS
K = 88
GRP = 6
N_PAD = 10112
BLK = 128
DT = 136


def _mesh():
    return plsc.VectorSubcoreMesh(
        core_axis_name="c", subcore_axis_name="s",
        num_cores=NC, num_subcores=NS)


def _sc_params():
    return pltpu.CompilerParams(use_tc_tiling_on_sc=False)


def _sc_degree(row3, n_pad):
    nchunk = row3.shape[1]
    rps = n_pad // NS

    @functools.partial(
        pl.kernel,
        out_type=jax.ShapeDtypeStruct((NC, n_pad), jnp.float32),
        mesh=_mesh(),
        compiler_params=_sc_params(),
        scratch_types=[
            pltpu.VMEM((nchunk, K), jnp.int32),
            pltpu.VMEM((K,), jnp.float32),
            pltpu.SemaphoreType.DMA,
            pltpu.VMEM_SHARED((n_pad,), jnp.float32),
        ],
    )
    def deg_kernel(row_hbm, ones_hbm, zeros_hbm, deg_hbm,
                   row_v, ones_v, dsem, deg_sp):
        c = lax.axis_index("c")
        s = lax.axis_index("s")
        t = c * NS + s

        zsl = pl.ds(s * rps, rps)
        pltpu.async_copy(ones_hbm, ones_v, dsem)
        pltpu.async_copy(zeros_hbm, deg_sp.at[zsl], dsem)
        pltpu.async_copy(row_hbm.at[t], row_v, dsem)
        pltpu.make_async_copy(ones_hbm, ones_v, dsem).wait()
        pltpu.make_async_copy(zeros_hbm, deg_sp.at[zsl], dsem).wait()
        pltpu.make_async_copy(row_hbm.at[t], row_v, dsem).wait()
        plsc.subcore_barrier()

        def body(i, carry):
            j0 = i * GRP
            for u in range(GRP):
                pltpu.async_copy(ones_v, deg_sp.at[row_v.at[j0 + u]], dsem,
                                 add=True)
            for u in range(GRP):
                pltpu.make_async_copy(ones_v, deg_sp.at[row_v.at[j0 + u]],
                                      dsem).wait()
            return carry
        lax.fori_loop(0, nchunk // GRP, body, 0)

        plsc.subcore_barrier()
        sl = pl.ds(s * rps, rps)
        pltpu.sync_copy(deg_sp.at[sl], deg_hbm.at[c].at[sl])

    return deg_kernel(row3, jnp.ones((K,), jnp.float32),
                      jnp.zeros((rps,), jnp.float32))


def _sc_spmm(z, row3, col3, n_pad, dt):
    nchunk = row3.shape[1]
    rps = n_pad // NS

    @functools.partial(
        pl.kernel,
        out_type=jax.ShapeDtypeStruct((NC, n_pad, dt), jnp.float32),
        mesh=_mesh(),
        compiler_params=_sc_params(),
        scratch_types=[
            pltpu.VMEM((nchunk, K), jnp.int32),
            pltpu.VMEM((nchunk, K), jnp.int32),
            pltpu.VMEM((2, K, dt), jnp.float32),
            pltpu.SemaphoreType.DMA((2,)),
            pltpu.SemaphoreType.DMA((2,)),
            pltpu.VMEM_SHARED((n_pad, dt), jnp.float32),
        ],
    )
    def spmm_kernel(z_hbm, row_hbm, col_hbm, zeros_hbm, acc_hbm,
                    row_v, col_v, gbuf, gsem, ssem, acc_sp):
        c = lax.axis_index("c")
        s = lax.axis_index("s")
        t = c * NS + s

        zsl = pl.ds(s * rps, rps)
        pltpu.async_copy(zeros_hbm, acc_sp.at[zsl], gsem.at[0])
        pltpu.async_copy(row_hbm.at[t], row_v, gsem.at[1])
        pltpu.async_copy(col_hbm.at[t], col_v, gsem.at[1])
        pltpu.make_async_copy(zeros_hbm, acc_sp.at[zsl], gsem.at[0]).wait()
        pltpu.make_async_copy(row_hbm.at[t], row_v, gsem.at[1]).wait()
        pltpu.make_async_copy(col_hbm.at[t], col_v, gsem.at[1]).wait()
        plsc.subcore_barrier()

        def start_gather(j, p):
            pltpu.async_copy(z_hbm.at[col_v.at[j]], gbuf.at[p], gsem.at[p])

        def wait_gather(j, p):
            pltpu.make_async_copy(z_hbm.at[col_v.at[j]], gbuf.at[p],
                                  gsem.at[p]).wait()

        def start_scatter(j, p):
            pltpu.async_copy(gbuf.at[p], acc_sp.at[row_v.at[j]],
                             ssem.at[p], add=True)

        def wait_scatter(j, p):
            pltpu.make_async_copy(gbuf.at[p], acc_sp.at[row_v.at[j]],
                                  ssem.at[p]).wait()

        start_gather(0, 0)

        def body(i, carry):
            j0 = 2 * i
            wait_gather(j0, 0)
            start_scatter(j0, 0)
            @pl.when(j0 > 0)
            def _():
                wait_scatter(j0 - 1, 1)
            start_gather(j0 + 1, 1)
            wait_gather(j0 + 1, 1)
            start_scatter(j0 + 1, 1)
            wait_scatter(j0, 0)
            @pl.when(j0 + 2 < nchunk)
            def _():
                start_gather(j0 + 2, 0)
            return carry
        lax.fori_loop(0, nchunk // 2, body, 0)
        wait_scatter(nchunk - 1, 1)

        plsc.subcore_barrier()
        sl = pl.ds(s * rps, rps)
        pltpu.sync_copy(acc_sp.at[sl], acc_hbm.at[c].at[sl])

    return spmm_kernel(z, row3, col3, jnp.zeros((rps, dt), jnp.float32))


def _tc_build_z(x, m, deg0, deg1, n_pad, d):
    def zk(x_ref, m_ref, d0_ref, d1_ref, z_ref):
        deg = d0_ref[...] + d1_ref[...]
        dis = jnp.where(deg > 0, lax.rsqrt(jnp.maximum(deg, 1.0)), 0.0)
        mv = m_ref[...]
        dm = dis * mv
        xv = x_ref[...]
        xv = jnp.where(jnp.isnan(xv), 0.0, xv)
        z_ref[...] = jnp.concatenate(
            [xv * dm, dis, dm, jnp.zeros((BLK, DT - d - 2), jnp.float32)],
            axis=1)

    return pl.pallas_call(
        zk,
        grid=(n_pad // BLK,),
        in_specs=[
            pl.BlockSpec((BLK, d), lambda i: (i, 0)),
            pl.BlockSpec((BLK, 1), lambda i: (i, 0)),
            pl.BlockSpec((BLK, 1), lambda i: (i, 0)),
            pl.BlockSpec((BLK, 1), lambda i: (i, 0)),
        ],
        out_specs=pl.BlockSpec((BLK, DT), lambda i: (i, 0)),
        out_shape=jax.ShapeDtypeStruct((n_pad, DT), jnp.float32),
    )(x, m, deg0, deg1)


def _tc_final(acc, deg0, deg1, W, b2, n, n_pad, d, dt):

    def fk(a_ref, d0_ref, d1_ref, w_ref, b_ref, o_ref):
        a0 = a_ref[0]
        a1 = a_ref[1]
        bsum = a0[:, :d] + a1[:, :d]
        asum = a0[:, d:d + 1] + a1[:, d:d + 1]
        csum = a0[:, d + 1:d + 2] + a1[:, d + 1:d + 2]
        deg = d0_ref[...] + d1_ref[...]
        dis = jnp.where(deg > 0, lax.rsqrt(jnp.maximum(deg, 1.0)), 0.0)
        nz = csum != 0.0
        scale = jnp.where(nz, dis * asum / jnp.where(nz, csum, 1.0), 0.0)
        ratio = bsum * scale
        out = lax.dot_general(ratio, w_ref[...], (((1,), (1,)), ((), ())),
                              preferred_element_type=jnp.float32)
        o_ref[...] = out + b_ref[...]

    return pl.pallas_call(
        fk,
        grid=(n_pad // BLK,),
        in_specs=[
            pl.BlockSpec((2, BLK, dt), lambda i: (0, i, 0)),
            pl.BlockSpec((BLK, 1), lambda i: (i, 0)),
            pl.BlockSpec((BLK, 1), lambda i: (i, 0)),
            pl.BlockSpec((d, d), lambda i: (0, 0)),
            pl.BlockSpec((1, d), lambda i: (0, 0)),
        ],
        out_specs=pl.BlockSpec((BLK, d), lambda i: (i, 0)),
        out_shape=jax.ShapeDtypeStruct((n, d), jnp.float32),
    )(acc, deg0, deg1, W, b2)


def kernel(x, edge_index, train_mask, W, b):
    n, d = x.shape
    e = edge_index.shape[1]
    n_pad = N_PAD

    nchunk = -(-e // (NW * K))
    nchunk = -(-nchunk // GRP) * GRP
    e_pad = NW * nchunk * K

    pad_len = e_pad - e
    pad_iota = jnp.arange(pad_len, dtype=jnp.int32)
    row = jnp.concatenate(
        [edge_index[0], n + pad_iota % (n_pad - n)])
    col = jnp.concatenate(
        [edge_index[1], pad_iota % n])
    row3 = row.reshape(NW, nchunk, K)
    col3 = col.reshape(NW, nchunk, K)

    deg2 = _sc_degree(row3, n_pad)
    deg0 = deg2[0].reshape(n_pad, 1)
    deg1 = deg2[1].reshape(n_pad, 1)

    z = _tc_build_z(x, train_mask, deg0, deg1, n_pad, d)

    acc = _sc_spmm(z, row3, col3, n_pad, DT)

    return _tc_final(acc, deg0, deg1, W, b.reshape(1, d), n, n_pad, d, DT)
